# baseline (speedup 1.0000x reference)
import functools

import jax
import jax.numpy as jnp
from jax import lax
from jax.experimental import pallas as pl
from jax.experimental.pallas import tpu as pltpu
from jax.experimental.pallas import tpu_sc as plsc

L = 16
NC, NS = 2, 16
NW = NC * NS


def _film_table_tc(cond_ref, w_ref, b_ref, tab_ref):
    ss = lax.dot_general(
        cond_ref[...], w_ref[...], (((1,), (1,)), ((), ())),
        preferred_element_type=jnp.float32,
    )
    ss = ss + b_ref[...]
    dim = tab_ref.shape[1] // 2
    col = lax.broadcasted_iota(jnp.int32, tab_ref.shape, 1)
    tab_ref[...] = ss + jnp.where(col < dim, 1.0, 0.0).astype(jnp.float32)


def _tc_affine(idx_ref, x_ref, tab_ref, o_ref):
    br = x_ref.shape[0]
    b_rows, two_dim = tab_ref.shape
    dim = two_dim // 2
    iot = lax.broadcasted_iota(jnp.int32, (br, b_rows), 1)
    oh = (idx_ref[...] == iot).astype(jnp.float32)
    gmb = lax.dot_general(oh, tab_ref[...], (((1,), (0,)), ((), ())),
                          preferred_element_type=jnp.float32)
    o_ref[...] = x_ref[...] * gmb[:, :dim] + gmb[:, dim:]


def _make_sc_kernel(n, n_proc, dim, b_rows, ch, nchunk):
    rpw = ch * nchunk
    ngrp = ch // L
    last_start = n_proc - ch
    assert nchunk % 2 == 0

    mesh = plsc.VectorSubcoreMesh(core_axis_name="c", subcore_axis_name="s")

    @functools.partial(
        pl.kernel,
        out_type=jax.ShapeDtypeStruct((n, dim), jnp.float32),
        mesh=mesh,
        scratch_types=[
            pltpu.VMEM((b_rows, 2 * dim), jnp.float32),
            pltpu.VMEM((2, ch, dim), jnp.float32),
            pltpu.VMEM((2, ch, dim), jnp.float32),
            pltpu.VMEM((rpw,), jnp.int32),
            pltpu.SemaphoreType.DMA,
            pltpu.SemaphoreType.DMA,
            pltpu.SemaphoreType.DMA,
            pltpu.SemaphoreType.DMA,
        ],
        compiler_params=pltpu.CompilerParams(needs_layout_passes=False),
    )
    def sc_kernel(x_hbm, idx_hbm, tab_hbm, out_hbm, tab_v, xb, ob, ib,
                  isem0, isem1, osem0, osem1):
        wid = lax.axis_index("s") * NC + lax.axis_index("c")
        base = wid * rpw
        ibase = jnp.minimum(base, n_proc - rpw)
        isems = (isem0, isem1)
        osems = (osem0, osem1)
        col = lax.iota(jnp.int32, 16)

        def start_of(c):
            return jnp.minimum(base + c * ch, last_start)

        def in_start(c, k):
            pltpu.async_copy(x_hbm.at[pl.ds(start_of(c), ch)], xb.at[k],
                             isems[k])

        def in_wait(c, k):
            pltpu.make_async_copy(x_hbm.at[pl.ds(start_of(c), ch)], xb.at[k],
                                  isems[k]).wait()

        def out_start(c, k):
            pltpu.async_copy(ob.at[k], out_hbm.at[pl.ds(start_of(c), ch)],
                             osems[k])

        def out_wait(c, k):
            pltpu.make_async_copy(ob.at[k], out_hbm.at[pl.ds(start_of(c), ch)],
                                  osems[k]).wait()

        def compute(c, k):
            ioff = start_of(c) - ibase
            nj = dim // L

            def grp_body(g, _):
                pos = ioff + g * L
                rv0 = plsc.load_gather(ib, [jnp.broadcast_to(pos, (L,))])
                rvf = plsc.load_gather(ib, [jnp.broadcast_to(pos + L - 1, (L,))])
                uniform = jnp.all(rv0 == rvf)

                @pl.when(uniform)
                def _uniform():
                    g1s = [plsc.load_gather(tab_v, [rv0, col + j * L])
                           for j in range(nj)]
                    bts = [plsc.load_gather(tab_v, [rv0, col + dim + j * L])
                           for j in range(nj)]

                    @plsc.parallel_loop(0, L, unroll=4)
                    def _rows(r):
                        row = g * L + r
                        for j in range(nj):
                            xv = xb[k, row, pl.ds(j * L, L)]
                            ob[k, row, pl.ds(j * L, L)] = xv * g1s[j] + bts[j]

                @pl.when(jnp.logical_not(uniform))
                def _mixed():
                    @plsc.parallel_loop(0, L, unroll=2)
                    def _rows(r):
                        row = g * L + r
                        rv = plsc.load_gather(
                            ib, [jnp.broadcast_to(ioff + row, (L,))])
                        for j in range(nj):
                            g1 = plsc.load_gather(tab_v, [rv, col + j * L])
                            bt = plsc.load_gather(tab_v, [rv, col + dim + j * L])
                            xv = xb[k, row, pl.ds(j * L, L)]
                            ob[k, row, pl.ds(j * L, L)] = xv * g1 + bt

                return 0

            lax.fori_loop(0, ngrp, grp_body, 0)

        pltpu.async_copy(idx_hbm.at[pl.ds(ibase, rpw)], ib, isem0)
        pltpu.sync_copy(tab_hbm, tab_v)
        pltpu.make_async_copy(idx_hbm.at[pl.ds(ibase, rpw)], ib, isem0).wait()
        in_start(0, 0)
        in_start(1, 1)

        def chunk_pair(i, _):
            for k in (0, 1):
                c = 2 * i + k
                in_wait(c, k)

                @pl.when(c >= 2)
                def _():
                    out_wait(c - 2, k)

                compute(c, k)
                out_start(c, k)

                @pl.when(c + 2 < nchunk)
                def _():
                    in_start(c + 2, k)

            return 0

        lax.fori_loop(0, nchunk // 2, chunk_pair, 0)
        out_wait(nchunk - 2, 0)
        out_wait(nchunk - 1, 1)

    return sc_kernel


def kernel(x, cond, n_index, W, b):
    n, dim = x.shape
    b_rows = cond.shape[0]

    tab = pl.pallas_call(
        _film_table_tc,
        out_shape=jax.ShapeDtypeStruct((b_rows, 2 * dim), jnp.float32),
    )(cond, W, b.reshape(1, 2 * dim))

    br = 1000
    n_sc = (n * 3 // 5) // br * br
    n_tc = n - n_sc
    ch = 160
    nchunk = -(-n_sc // (NW * ch))
    nchunk += nchunk % 2
    assert n_sc >= ch and (n_sc - ch) % 8 == 0 and dim % L == 0
    assert ch % L == 0 and n_tc % br == 0 and n_sc % br == 0

    idx32 = n_index.astype(jnp.int32)
    sc = _make_sc_kernel(n, n_sc, dim, b_rows, ch, nchunk)
    out_sc = sc(x, idx32, tab)

    nb = n_sc // br
    out_tc = pl.pallas_call(
        _tc_affine,
        grid=(n_tc // br,),
        in_specs=[
            pl.BlockSpec((br, 1), lambda i: (nb + i, 0)),
            pl.BlockSpec((br, dim), lambda i: (nb + i, 0)),
            pl.BlockSpec((b_rows, 2 * dim), lambda i: (0, 0)),
        ],
        out_specs=pl.BlockSpec((br, dim), lambda i: (i, 0)),
        out_shape=jax.ShapeDtypeStruct((n_tc, dim), jnp.float32),
    )(idx32.reshape(n, 1), x, tab)

    return lax.dynamic_update_slice(out_sc, out_tc, (n_sc, 0))


# speedup vs baseline: 1.9452x; 1.9452x over previous
import functools

import jax
import jax.numpy as jnp
from jax import lax
from jax.experimental import pallas as pl
from jax.experimental.pallas import tpu as pltpu
from jax.experimental.pallas import tpu_sc as plsc

L = 16
NC, NS = 2, 16
NW = NC * NS


def _film_table_tc(cond_ref, w_ref, b_ref, tab_ref):
    ss = lax.dot_general(
        cond_ref[...], w_ref[...], (((1,), (1,)), ((), ())),
        preferred_element_type=jnp.float32,
    )
    ss = ss + b_ref[...]
    dim = tab_ref.shape[1] // 2
    col = lax.broadcasted_iota(jnp.int32, tab_ref.shape, 1)
    tab_ref[...] = ss + jnp.where(col < dim, 1.0, 0.0).astype(jnp.float32)


def _make_sc_kernel(n, dim, b_rows, ch, nchunk):
    rpw = ch * nchunk
    ngrp = ch // L
    last_start = n - ch
    assert nchunk % 2 == 0

    mesh = plsc.VectorSubcoreMesh(core_axis_name="c", subcore_axis_name="s")

    @functools.partial(
        pl.kernel,
        out_type=jax.ShapeDtypeStruct((n, dim), jnp.float32),
        mesh=mesh,
        scratch_types=[
            pltpu.VMEM((b_rows, 2 * dim), jnp.float32),
            pltpu.VMEM((2, ch, dim), jnp.float32),
            pltpu.VMEM((2, ch, dim), jnp.float32),
            pltpu.VMEM((rpw,), jnp.int32),
            pltpu.SemaphoreType.DMA,
            pltpu.SemaphoreType.DMA,
            pltpu.SemaphoreType.DMA,
            pltpu.SemaphoreType.DMA,
        ],
        compiler_params=pltpu.CompilerParams(needs_layout_passes=False),
    )
    def sc_kernel(x_hbm, idx_hbm, tab_hbm, out_hbm, tab_v, xb, ob, ib,
                  isem0, isem1, osem0, osem1):
        wid = lax.axis_index("s") * NC + lax.axis_index("c")
        base = wid * rpw
        ibase = jnp.minimum(base, n - rpw)
        isems = (isem0, isem1)
        osems = (osem0, osem1)
        col = lax.iota(jnp.int32, 16)

        def start_of(c):
            return jnp.minimum(base + c * ch, last_start)

        def in_start(c, k):
            pltpu.async_copy(x_hbm.at[pl.ds(start_of(c), ch)], xb.at[k],
                             isems[k])

        def in_wait(c, k):
            pltpu.make_async_copy(x_hbm.at[pl.ds(start_of(c), ch)], xb.at[k],
                                  isems[k]).wait()

        def out_start(c, k):
            pltpu.async_copy(ob.at[k], out_hbm.at[pl.ds(start_of(c), ch)],
                             osems[k])

        def out_wait(c, k):
            pltpu.make_async_copy(ob.at[k], out_hbm.at[pl.ds(start_of(c), ch)],
                                  osems[k]).wait()

        def compute(c, k):
            ioff = start_of(c) - ibase
            nj = dim // L

            cv0 = plsc.load_gather(ib, [jnp.broadcast_to(ioff, (L,))])
            cvf = plsc.load_gather(ib, [jnp.broadcast_to(ioff + ch - 1, (L,))])
            chunk_uniform = jnp.all(cv0 == cvf)

            @pl.when(chunk_uniform)
            def _chunk_uniform():
                g1s = [plsc.load_gather(tab_v, [cv0, col + j * L])
                       for j in range(nj)]
                bts = [plsc.load_gather(tab_v, [cv0, col + dim + j * L])
                       for j in range(nj)]

                @plsc.parallel_loop(0, ch, unroll=4)
                def _rows(row):
                    for j in range(nj):
                        xv = xb[k, row, pl.ds(j * L, L)]
                        ob[k, row, pl.ds(j * L, L)] = xv * g1s[j] + bts[j]

            @pl.when(jnp.logical_not(chunk_uniform))
            def _chunk_groups():
                _per_group(c, k, ioff, nj)

        def _per_group(c, k, ioff, nj):
            def grp_body(g, _):
                pos = ioff + g * L
                rv0 = plsc.load_gather(ib, [jnp.broadcast_to(pos, (L,))])
                rvf = plsc.load_gather(ib, [jnp.broadcast_to(pos + L - 1, (L,))])
                uniform = jnp.all(rv0 == rvf)

                @pl.when(uniform)
                def _uniform():
                    g1s = [plsc.load_gather(tab_v, [rv0, col + j * L])
                           for j in range(nj)]
                    bts = [plsc.load_gather(tab_v, [rv0, col + dim + j * L])
                           for j in range(nj)]

                    @plsc.parallel_loop(0, L, unroll=4)
                    def _rows(r):
                        row = g * L + r
                        for j in range(nj):
                            xv = xb[k, row, pl.ds(j * L, L)]
                            ob[k, row, pl.ds(j * L, L)] = xv * g1s[j] + bts[j]

                @pl.when(jnp.logical_not(uniform))
                def _mixed():
                    @plsc.parallel_loop(0, L, unroll=2)
                    def _rows(r):
                        row = g * L + r
                        rv = plsc.load_gather(
                            ib, [jnp.broadcast_to(ioff + row, (L,))])
                        for j in range(nj):
                            g1 = plsc.load_gather(tab_v, [rv, col + j * L])
                            bt = plsc.load_gather(tab_v, [rv, col + dim + j * L])
                            xv = xb[k, row, pl.ds(j * L, L)]
                            ob[k, row, pl.ds(j * L, L)] = xv * g1 + bt

                return 0

            lax.fori_loop(0, ngrp, grp_body, 0)

        pltpu.async_copy(idx_hbm.at[pl.ds(ibase, rpw)], ib, isem0)
        pltpu.sync_copy(tab_hbm, tab_v)
        pltpu.make_async_copy(idx_hbm.at[pl.ds(ibase, rpw)], ib, isem0).wait()
        in_start(0, 0)
        in_start(1, 1)

        def chunk_pair(i, _):
            for k in (0, 1):
                c = 2 * i + k
                in_wait(c, k)

                @pl.when(c >= 2)
                def _():
                    out_wait(c - 2, k)

                compute(c, k)
                out_start(c, k)

                @pl.when(c + 2 < nchunk)
                def _():
                    in_start(c + 2, k)

            return 0

        lax.fori_loop(0, nchunk // 2, chunk_pair, 0)
        out_wait(nchunk - 2, 0)
        out_wait(nchunk - 1, 1)

    return sc_kernel


def kernel(x, cond, n_index, W, b):
    n, dim = x.shape
    b_rows = cond.shape[0]

    tab = pl.pallas_call(
        _film_table_tc,
        out_shape=jax.ShapeDtypeStruct((b_rows, 2 * dim), jnp.float32),
    )(cond, W, b.reshape(1, 2 * dim))

    ch = 176
    nchunk = -(-n // (NW * ch))
    assert n >= ch and (n - ch) % 8 == 0 and dim % L == 0 and ch % L == 0

    idx32 = n_index.astype(jnp.int32)
    sc = _make_sc_kernel(n, dim, b_rows, ch, nchunk)
    return sc(x, idx32, tab)
